# baseline (speedup 1.0000x reference)
import functools

import jax
import jax.numpy as jnp
from jax import lax
from jax.experimental import pallas as pl
from jax.experimental.pallas import tpu as pltpu
from jax.experimental.pallas import tpu_sc as plsc

_NU, _NE, _NK, _D = 50000, 50000, 2048, 64
_B = 1024
_R = 256
_SPREAD = 64

_NUP = 50176
_NEP = 50176
_N1 = 100352
_N2 = 53248
_NA = 51200
_NKP = 2176
_EH = 802816
_E1 = 1605632
_E2 = 425984
_EA = 212992


def _sc_mesh():
    return plsc.VectorSubcoreMesh(core_axis_name="c", subcore_axis_name="s")


_SC_PARAMS = pltpu.CompilerParams(use_tc_tiling_on_sc=False)
_SC_PARAMS_NL = pltpu.CompilerParams(use_tc_tiling_on_sc=False,
                                     needs_layout_passes=False)


def _chunks(total, step=1024):
    out, o = [], 0
    while o < total:
        c = min(step, total - o)
        out.append((o, c))
        o += c
    return out


def _after(x, dep):
    return lax.optimization_barrier((x, dep))[0]


def _make_deg(n_p, e_pad):
    nrow = n_p // 16
    nz = nrow // 16
    nblk = e_pad // 32 // _B
    step = {6272: 896, 3328: 832}[nrow]
    nmerge = nrow // step

    def body(row_hbm, zrow_hbm, iota_hbm, out_hbm, idx_v, mi_v, st_v, hist_v,
             deg_sp):
        cid = lax.axis_index("c")
        sid = lax.axis_index("s")
        wid = sid * 2 + cid
        pltpu.sync_copy(zrow_hbm, st_v)
        pltpu.sync_copy(st_v, deg_sp.at[pl.ds(sid * nz, nz)])

        def zblk(k, carry):
            hist_v[k] = jnp.zeros((16,), jnp.int32)
            return carry

        lax.fori_loop(0, nrow, zblk, 0)
        plsc.subcore_barrier()

        def blk(k, carry):
            e0 = wid * (e_pad // 32) + k * _B
            pltpu.sync_copy(row_hbm.at[pl.ds(e0, _B)], idx_v)

            def inner(j, c2):
                idx16 = idx_v[pl.ds(j * 16, 16)]
                cnt, last = plsc.scan_count(idx16)
                plsc.addupdate_scatter(
                    hist_v,
                    [lax.shift_right_logical(idx16, 4),
                     lax.bitwise_and(idx16, 15)],
                    cnt, mask=last)
                return c2

            lax.fori_loop(0, _B // 16, inner, 0)
            return carry

        lax.fori_loop(0, nblk, blk, 0)
        for m in range(nmerge):
            pltpu.sync_copy(iota_hbm.at[pl.ds(m * step, step)], mi_v)
            pltpu.sync_copy(hist_v.at[pl.ds(m * step, step)],
                            deg_sp.at[mi_v], add=True)
        plsc.subcore_barrier()
        pltpu.sync_copy(deg_sp.at[pl.ds(sid * nz, nz)], st_v)
        pltpu.sync_copy(st_v, out_hbm.at[cid, pl.ds(sid * nz, nz)])

    return pl.kernel(
        body,
        out_type=jax.ShapeDtypeStruct((2, nrow, 16), jnp.int32),
        mesh=_sc_mesh(),
        compiler_params=_SC_PARAMS_NL,
        scratch_types=[
            pltpu.VMEM((_B,), jnp.int32),
            pltpu.VMEM((step,), jnp.int32),
            pltpu.VMEM((nz, 16), jnp.int32),
            pltpu.VMEM((nrow, 16), jnp.int32),
            pltpu.VMEM_SHARED((nrow, 16), jnp.int32),
        ],
    )


def _scat_shard_pass(nr, nblk, e_pad, sid, shard, col_hbm, row_hbm, out_view,
                     col_v, row_v, rows_v, zb_v, wb_v, sem, acc_sp, tab_hbm):
    for (o, c) in _chunks(nr):
        pltpu.sync_copy(zb_v.at[pl.ds(0, c)],
                        acc_sp.at[pl.ds(sid * nr + o, c)])
    plsc.subcore_barrier()

    def blk(k, carry):
        e0 = sid * (e_pad // 16) + k * _B
        pltpu.sync_copy(col_hbm.at[pl.ds(shard * e_pad + e0, _B)], col_v)
        pltpu.sync_copy(row_hbm.at[pl.ds(e0, _B)], row_v)
        pltpu.async_copy(tab_hbm.at[col_v], rows_v, sem).wait()
        pltpu.sync_copy(rows_v, acc_sp.at[row_v], add=True)
        return carry

    lax.fori_loop(0, nblk, blk, 0)
    plsc.subcore_barrier()
    for (o, c) in _chunks(nr):
        pltpu.sync_copy(acc_sp.at[pl.ds(sid * nr + o, c)],
                        wb_v.at[pl.ds(0, c)])
        pltpu.sync_copy(wb_v.at[pl.ds(0, c)],
                        out_view.at[pl.ds(sid * nr + o, c)])


def _make_scat(n_acc, e_pad):
    nr = n_acc // 16
    nblk = e_pad // 16 // _B

    def body(tab_hbm, col4_hbm, row_hbm, zrows_hbm, out_hbm,
             col_v, row_v, rows_v, zb_v, wb_v, sem, acc_sp):
        cid = lax.axis_index("c")
        sid = lax.axis_index("s")
        pltpu.sync_copy(zrows_hbm, zb_v)
        for cc in range(2):
            shard = cid * 2 + cc
            _scat_shard_pass(nr, nblk, e_pad, sid, shard, col4_hbm, row_hbm,
                             out_hbm.at[shard], col_v, row_v, rows_v, zb_v,
                             wb_v, sem, acc_sp, tab_hbm)

    return pl.kernel(
        body,
        out_type=jax.ShapeDtypeStruct((4, n_acc, 16), jnp.float32),
        mesh=_sc_mesh(),
        compiler_params=_SC_PARAMS,
        scratch_types=[
            pltpu.VMEM((_B,), jnp.int32),
            pltpu.VMEM((_B,), jnp.int32),
            pltpu.VMEM((_B, 16), jnp.float32),
            pltpu.VMEM((1024, 16), jnp.float32),
            pltpu.VMEM((1024, 16), jnp.float32),
            pltpu.SemaphoreType.DMA,
            pltpu.VMEM_SHARED((n_acc, 16), jnp.float32),
        ],
    )


def _make_scat_halves(n_acc, e_pad_h):
    nr = n_acc // 16
    nblk = e_pad_h // 16 // _B

    def body(tab_hbm, colA_hbm, rowA_hbm, colB_hbm, rowB_hbm, zrows_hbm,
             out_hbm, col_v, row_v, rows_v, zb_v, wb_v, sem, acc_sp):
        cid = lax.axis_index("c")
        sid = lax.axis_index("s")
        pltpu.sync_copy(zrows_hbm, zb_v)
        for h, (c_hbm, r_hbm) in enumerate(((colA_hbm, rowA_hbm),
                                            (colB_hbm, rowB_hbm))):
            for cc in range(2):
                shard = cid * 2 + cc
                _scat_shard_pass(nr, nblk, e_pad_h, sid, shard, c_hbm, r_hbm,
                                 out_hbm.at[shard, h], col_v, row_v, rows_v,
                                 zb_v, wb_v, sem, acc_sp, tab_hbm)

    return pl.kernel(
        body,
        out_type=jax.ShapeDtypeStruct((4, 2, n_acc, 16), jnp.float32),
        mesh=_sc_mesh(),
        compiler_params=_SC_PARAMS,
        scratch_types=[
            pltpu.VMEM((_B,), jnp.int32),
            pltpu.VMEM((_B,), jnp.int32),
            pltpu.VMEM((_B, 16), jnp.float32),
            pltpu.VMEM((1024, 16), jnp.float32),
            pltpu.VMEM((1024, 16), jnp.float32),
            pltpu.SemaphoreType.DMA,
            pltpu.VMEM_SHARED((n_acc, 16), jnp.float32),
        ],
    )


def _prep_body(feats_ref, d0_ref, d1_ref, fp_ref):
    deg = (d0_ref[...] + d1_ref[...]).astype(jnp.float32)
    dinv = 1.0 / (jnp.sqrt(deg) + 1e-8)
    x = feats_ref[...] * dinv
    for c in range(4):
        fp_ref[c] = x[:, c * 16:(c + 1) * 16]


def _make_prep(n_p):
    return pl.pallas_call(
        _prep_body,
        grid=(n_p // _R,),
        in_specs=[
            pl.BlockSpec((_R, _D), lambda i: (i, 0)),
            pl.BlockSpec((_R, 1), lambda i: (i, 0)),
            pl.BlockSpec((_R, 1), lambda i: (i, 0)),
        ],
        out_specs=pl.BlockSpec((4, _R, 16), lambda i: (0, i, 0)),
        out_shape=jax.ShapeDtypeStruct((4, n_p, 16), jnp.float32),
    )


def _layer_body(damp, last, s_ref, d0_ref, d1_ref, acc_ref, *out_refs):
    deg = (d0_ref[...] + d1_ref[...]).astype(jnp.float32)
    dinv = 1.0 / (jnp.sqrt(deg) + 1e-8)
    x = jnp.concatenate([s_ref[c] for c in range(4)], axis=1)
    f = x * dinv * (1.0 / damp)
    l2 = jnp.sqrt(jnp.sum(f * f, axis=1, keepdims=True))
    out_refs[0][...] = acc_ref[...] + f / jnp.maximum(l2, 1e-12)
    if not last:
        fn = f * dinv
        for c in range(4):
            out_refs[1][c] = fn[:, c * 16:(c + 1) * 16]


def _make_layer(n_p, damp, last):
    out_specs = [pl.BlockSpec((_R, _D), lambda i: (i, 0))]
    out_shape = [jax.ShapeDtypeStruct((n_p, _D), jnp.float32)]
    if not last:
        out_specs.append(pl.BlockSpec((4, _R, 16), lambda i: (0, i, 0)))
        out_shape.append(jax.ShapeDtypeStruct((4, n_p, 16), jnp.float32))
    return pl.pallas_call(
        functools.partial(_layer_body, float(damp), last),
        grid=(n_p // _R,),
        in_specs=[
            pl.BlockSpec((4, _R, 16), lambda i: (0, i, 0)),
            pl.BlockSpec((_R, 1), lambda i: (i, 0)),
            pl.BlockSpec((_R, 1), lambda i: (i, 0)),
            pl.BlockSpec((_R, _D), lambda i: (i, 0)),
        ],
        out_specs=out_specs,
        out_shape=out_shape,
    )


_RM = 400


def _mlp_body(au_ref, ak_ref, sa_ref, de0_ref, de1_ref, w_ref, b_ref,
              out_ref):
    deg = (de0_ref[...] + de1_ref[...]).astype(jnp.float32) + 1e-8
    ea = jnp.concatenate([sa_ref[c] for c in range(4)], axis=1) / deg
    h1 = au_ref[...]
    h2 = ak_ref[...] + ea
    w = w_ref[...]
    y = (jnp.dot(h1, w[:_D], preferred_element_type=jnp.float32)
         + jnp.dot(h2, w[_D:], preferred_element_type=jnp.float32)
         + b_ref[...])
    out_ref[...] = jnp.maximum(y, 0.0)


def _make_mlp():
    return pl.pallas_call(
        _mlp_body,
        grid=(_NE // _RM,),
        in_specs=[
            pl.BlockSpec((_RM, _D), lambda i: (i, 0)),
            pl.BlockSpec((_RM, _D), lambda i: (i, 0)),
            pl.BlockSpec((4, _RM, 16), lambda i: (0, i, 0)),
            pl.BlockSpec((_RM, 1), lambda i: (i, 0)),
            pl.BlockSpec((_RM, 1), lambda i: (i, 0)),
            pl.BlockSpec((2 * _D, _D), lambda i: (0, 0)),
            pl.BlockSpec((1, _D), lambda i: (0, 0)),
        ],
        out_specs=pl.BlockSpec((_RM, _D), lambda i: (i, 0)),
        out_shape=jax.ShapeDtypeStruct((_NE, _D), jnp.float32),
    )


def _pad_idx(x, total, dummy_base):
    pad = total - x.shape[0]
    spread = jnp.arange(pad, dtype=jnp.int32) % _SPREAD
    return jnp.concatenate([x, dummy_base + spread])


def _col4(col, total, n_tab):
    col = _pad_idx(col[0], total, col[1]) if isinstance(col, tuple) else col
    return (col[None, :]
            + (jnp.arange(4, dtype=jnp.int32) * n_tab)[:, None]).reshape(-1)


_DEG1 = _make_deg(_N1, _E1)
_DEG2 = _make_deg(_N2, _E2)
_SCAT1 = _make_scat_halves(_NUP, _EH)
_SCAT2 = _make_scat(_N2, _E2)
_SCATA = _make_scat(_NA, _EA)
_PREP1 = _make_prep(_N1)
_PREP2 = _make_prep(_N2)
_LAYER1 = (_make_layer(_N1, 2, False), _make_layer(_N1, 3, True))
_LAYER2 = (_make_layer(_N2, 2, False), _make_layer(_N2, 3, True))
_MLP = _make_mlp()


def kernel(users_feature, exercises_feature, knowledge_feature, W_mlp, b_mlp,
           ue_src, ue_dst, ek_src, ek_dst):
    zrows = jnp.zeros((1024, 16), jnp.float32)

    rowA = _pad_idx(ue_src, _EH, _NU)
    colA4 = _col4((ue_dst + _NUP, _NU), _EH, _N1)
    rowB = _pad_idx(ue_dst, _EH, _NE)
    colB4 = _col4((ue_src, _NU), _EH, _N1)
    rowcat1 = _pad_idx(jnp.concatenate([ue_src, ue_dst + _NUP]), _E1, _NU)
    feats1 = jnp.concatenate([
        jnp.pad(users_feature, ((0, _NUP - _NU), (0, 0))),
        jnp.pad(exercises_feature, ((0, _NEP - _NE), (0, 0))),
    ])

    deg1 = _DEG1(rowcat1, jnp.zeros((_N1 // 256, 16), jnp.int32),
                 jnp.arange(_N1 // 16, dtype=jnp.int32))
    d0 = deg1[0].reshape(_N1, 1)
    d1 = deg1[1].reshape(_N1, 1)
    fp = _PREP1(feats1, d0, d1)
    s = _SCAT1(fp.reshape(4 * _N1, 16), colA4, rowA, colB4, rowB,
               zrows).reshape(4, _N1, 16)
    acc1, fp = _LAYER1[0](s, d0, d1, feats1)
    s = _SCAT1(fp.reshape(4 * _N1, 16), colA4, rowA, colB4, rowB,
               zrows).reshape(4, _N1, 16)
    (acc1,) = _LAYER1[1](s, d0, d1, acc1)

    n2 = _NE + _NK
    row2 = _pad_idx(jnp.concatenate([ek_src, ek_dst + _NE]), _E2, n2)
    col2 = jnp.concatenate([ek_dst + _NE, ek_src])
    col24 = _col4((col2, n2), _E2, _N2)
    row2 = _after(row2, acc1)
    feats2 = jnp.concatenate([
        exercises_feature, knowledge_feature,
        jnp.zeros((_N2 - n2, _D), jnp.float32),
    ])

    deg2 = _DEG2(row2, jnp.zeros((_N2 // 256, 16), jnp.int32),
                 jnp.arange(_N2 // 16, dtype=jnp.int32))
    e0 = deg2[0].reshape(_N2, 1)
    e1 = deg2[1].reshape(_N2, 1)
    fp2 = _PREP2(feats2, e0, e1)
    s2 = _SCAT2(fp2.reshape(4 * _N2, 16), col24, row2, zrows)
    acc2, fp2 = _LAYER2[0](s2, e0, e1, feats2)
    s2 = _SCAT2(fp2.reshape(4 * _N2, 16), col24, row2, zrows)
    (acc2,) = _LAYER2[1](s2, e0, e1, acc2)

    krep = acc2[_NE:_NE + _NK]
    krep4 = jnp.pad(krep.reshape(_NK, 4, 16).transpose(1, 0, 2),
                    ((0, 0), (0, _NKP - _NK), (0, 0)))
    rowa = _pad_idx(ek_src, _EA, _NE)
    rowa = _after(rowa, acc2)
    cola4 = _col4((ek_dst, _NK), _EA, _NKP)
    sa = _SCATA(krep4.reshape(4 * _NKP, 16), cola4, rowa, zrows)

    a_u = lax.slice(acc1, (_NUP, 0), (_NUP + _NE, _D))
    a_k = acc2[:_NE]
    out_e = _MLP(a_u, a_k, sa, e0, e1, W_mlp, b_mlp.reshape(1, _D))
    return jnp.concatenate([acc1[:_NU], out_e], axis=0)


# speedup vs baseline: 1.1726x; 1.1726x over previous
import functools

import jax
import jax.numpy as jnp
from jax import lax
from jax.experimental import pallas as pl
from jax.experimental.pallas import tpu as pltpu
from jax.experimental.pallas import tpu_sc as plsc

_NU, _NE, _NK, _D = 50000, 50000, 2048, 64
_B = 1024
_R = 256
_SPREAD = 64

_NUP = 50176
_NEP = 50176
_N1 = 100352
_N2 = 53248
_NA = 51200
_NKP = 2176
_EH = 802816
_E1 = 1605632
_E2 = 425984
_EA = 212992


def _sc_mesh():
    return plsc.VectorSubcoreMesh(core_axis_name="c", subcore_axis_name="s")


_SC_PARAMS = pltpu.CompilerParams(use_tc_tiling_on_sc=False)
_SC_PARAMS_NL = pltpu.CompilerParams(use_tc_tiling_on_sc=False,
                                     needs_layout_passes=False)


def _chunks(total, step=1024):
    out, o = [], 0
    while o < total:
        c = min(step, total - o)
        out.append((o, c))
        o += c
    return out


def _after(x, dep):
    return lax.optimization_barrier((x, dep))[0]


def _make_deg(n_p, e_pad):
    nrow = n_p // 16
    nz = nrow // 16
    nblk = e_pad // 32 // _B
    step = {6272: 896, 3328: 832}[nrow]
    nmerge = nrow // step

    def body(row_hbm, zrow_hbm, iota_hbm, out_hbm, idx_v, mi_v, st_v, hist_v,
             deg_sp):
        cid = lax.axis_index("c")
        sid = lax.axis_index("s")
        wid = sid * 2 + cid
        pltpu.sync_copy(zrow_hbm, st_v)
        pltpu.sync_copy(st_v, deg_sp.at[pl.ds(sid * nz, nz)])

        def zblk(k, carry):
            hist_v[k] = jnp.zeros((16,), jnp.int32)
            return carry

        lax.fori_loop(0, nrow, zblk, 0)
        plsc.subcore_barrier()

        def blk(k, carry):
            e0 = wid * (e_pad // 32) + k * _B
            pltpu.sync_copy(row_hbm.at[pl.ds(e0, _B)], idx_v)

            def inner(j, c2):
                idx16 = idx_v[pl.ds(j * 16, 16)]
                cnt, last = plsc.scan_count(idx16)
                plsc.addupdate_scatter(
                    hist_v,
                    [lax.shift_right_logical(idx16, 4),
                     lax.bitwise_and(idx16, 15)],
                    cnt, mask=last)
                return c2

            lax.fori_loop(0, _B // 16, inner, 0)
            return carry

        lax.fori_loop(0, nblk, blk, 0)
        for m in range(nmerge):
            pltpu.sync_copy(iota_hbm.at[pl.ds(m * step, step)], mi_v)
            pltpu.sync_copy(hist_v.at[pl.ds(m * step, step)],
                            deg_sp.at[mi_v], add=True)
        plsc.subcore_barrier()
        pltpu.sync_copy(deg_sp.at[pl.ds(sid * nz, nz)], st_v)
        pltpu.sync_copy(st_v, out_hbm.at[cid, pl.ds(sid * nz, nz)])

    return pl.kernel(
        body,
        out_type=jax.ShapeDtypeStruct((2, nrow, 16), jnp.int32),
        mesh=_sc_mesh(),
        compiler_params=_SC_PARAMS_NL,
        scratch_types=[
            pltpu.VMEM((_B,), jnp.int32),
            pltpu.VMEM((step,), jnp.int32),
            pltpu.VMEM((nz, 16), jnp.int32),
            pltpu.VMEM((nrow, 16), jnp.int32),
            pltpu.VMEM_SHARED((nrow, 16), jnp.int32),
        ],
    )


def _scat_shard_pass(nr, nblk, e_pad, sid, shard, col_hbm, row_hbm, out_view,
                     col_v, row_v, rows_v, zb_v, wb_v, sem, acc_sp, tab_hbm):
    for (o, c) in _chunks(nr):
        pltpu.sync_copy(zb_v.at[pl.ds(0, c)],
                        acc_sp.at[pl.ds(sid * nr + o, c)])
    plsc.subcore_barrier()
    ebase = sid * (e_pad // 16)

    def blk2(kk, carry):
        gds = []
        for p in (0, 1):
            e0 = ebase + (2 * kk + p) * _B
            pltpu.sync_copy(col_hbm.at[pl.ds(shard * e_pad + e0, _B)],
                            col_v.at[p])
            gds.append(pltpu.async_copy(tab_hbm.at[col_v.at[p]],
                                        rows_v.at[p], sem))
        for p in (0, 1):
            e0 = ebase + (2 * kk + p) * _B
            pltpu.sync_copy(row_hbm.at[pl.ds(e0, _B)], row_v.at[p])
        for p in (0, 1):
            gds[p].wait()
            pltpu.sync_copy(rows_v.at[p], acc_sp.at[row_v.at[p]], add=True)
        return carry

    lax.fori_loop(0, nblk // 2, blk2, 0)
    if nblk % 2:
        e0 = ebase + (nblk - 1) * _B
        pltpu.sync_copy(col_hbm.at[pl.ds(shard * e_pad + e0, _B)],
                        col_v.at[0])
        pltpu.sync_copy(row_hbm.at[pl.ds(e0, _B)], row_v.at[0])
        pltpu.async_copy(tab_hbm.at[col_v.at[0]], rows_v.at[0], sem).wait()
        pltpu.sync_copy(rows_v.at[0], acc_sp.at[row_v.at[0]], add=True)
    plsc.subcore_barrier()
    for (o, c) in _chunks(nr):
        pltpu.sync_copy(acc_sp.at[pl.ds(sid * nr + o, c)],
                        wb_v.at[pl.ds(0, c)])
        pltpu.sync_copy(wb_v.at[pl.ds(0, c)],
                        out_view.at[pl.ds(sid * nr + o, c)])


def _make_scat(n_acc, e_pad):
    nr = n_acc // 16
    nblk = e_pad // 16 // _B

    def body(tab_hbm, col4_hbm, row_hbm, zrows_hbm, out_hbm,
             col_v, row_v, rows_v, zb_v, wb_v, sem, acc_sp):
        cid = lax.axis_index("c")
        sid = lax.axis_index("s")
        pltpu.sync_copy(zrows_hbm, zb_v)
        for cc in range(2):
            shard = cid * 2 + cc
            _scat_shard_pass(nr, nblk, e_pad, sid, shard, col4_hbm, row_hbm,
                             out_hbm.at[shard], col_v, row_v, rows_v, zb_v,
                             wb_v, sem, acc_sp, tab_hbm)

    return pl.kernel(
        body,
        out_type=jax.ShapeDtypeStruct((4, n_acc, 16), jnp.float32),
        mesh=_sc_mesh(),
        compiler_params=_SC_PARAMS,
        scratch_types=[
            pltpu.VMEM((2, _B), jnp.int32),
            pltpu.VMEM((2, _B), jnp.int32),
            pltpu.VMEM((2, _B, 16), jnp.float32),
            pltpu.VMEM((1024, 16), jnp.float32),
            pltpu.VMEM((1024, 16), jnp.float32),
            pltpu.SemaphoreType.DMA,
            pltpu.VMEM_SHARED((n_acc, 16), jnp.float32),
        ],
    )


def _make_scat_halves(n_acc, e_pad_h):
    nr = n_acc // 16
    nblk = e_pad_h // 16 // _B

    def body(tab_hbm, colA_hbm, rowA_hbm, colB_hbm, rowB_hbm, zrows_hbm,
             out_hbm, col_v, row_v, rows_v, zb_v, wb_v, sem, acc_sp):
        cid = lax.axis_index("c")
        sid = lax.axis_index("s")
        pltpu.sync_copy(zrows_hbm, zb_v)
        for h, (c_hbm, r_hbm) in enumerate(((colA_hbm, rowA_hbm),
                                            (colB_hbm, rowB_hbm))):
            for cc in range(2):
                shard = cid * 2 + cc
                _scat_shard_pass(nr, nblk, e_pad_h, sid, shard, c_hbm, r_hbm,
                                 out_hbm.at[shard, h], col_v, row_v, rows_v,
                                 zb_v, wb_v, sem, acc_sp, tab_hbm)

    return pl.kernel(
        body,
        out_type=jax.ShapeDtypeStruct((4, 2, n_acc, 16), jnp.float32),
        mesh=_sc_mesh(),
        compiler_params=_SC_PARAMS,
        scratch_types=[
            pltpu.VMEM((2, _B), jnp.int32),
            pltpu.VMEM((2, _B), jnp.int32),
            pltpu.VMEM((2, _B, 16), jnp.float32),
            pltpu.VMEM((1024, 16), jnp.float32),
            pltpu.VMEM((1024, 16), jnp.float32),
            pltpu.SemaphoreType.DMA,
            pltpu.VMEM_SHARED((n_acc, 16), jnp.float32),
        ],
    )


def _prep_body(feats_ref, d0_ref, d1_ref, fp_ref):
    deg = (d0_ref[...] + d1_ref[...]).astype(jnp.float32)
    dinv = 1.0 / (jnp.sqrt(deg) + 1e-8)
    x = feats_ref[...] * dinv
    for c in range(4):
        fp_ref[c] = x[:, c * 16:(c + 1) * 16]


def _make_prep(n_p):
    return pl.pallas_call(
        _prep_body,
        grid=(n_p // _R,),
        in_specs=[
            pl.BlockSpec((_R, _D), lambda i: (i, 0)),
            pl.BlockSpec((_R, 1), lambda i: (i, 0)),
            pl.BlockSpec((_R, 1), lambda i: (i, 0)),
        ],
        out_specs=pl.BlockSpec((4, _R, 16), lambda i: (0, i, 0)),
        out_shape=jax.ShapeDtypeStruct((4, n_p, 16), jnp.float32),
    )


def _layer_body(damp, last, s_ref, d0_ref, d1_ref, acc_ref, *out_refs):
    deg = (d0_ref[...] + d1_ref[...]).astype(jnp.float32)
    dinv = 1.0 / (jnp.sqrt(deg) + 1e-8)
    x = jnp.concatenate([s_ref[c] for c in range(4)], axis=1)
    f = x * dinv * (1.0 / damp)
    l2 = jnp.sqrt(jnp.sum(f * f, axis=1, keepdims=True))
    out_refs[0][...] = acc_ref[...] + f / jnp.maximum(l2, 1e-12)
    if not last:
        fn = f * dinv
        for c in range(4):
            out_refs[1][c] = fn[:, c * 16:(c + 1) * 16]


def _make_layer(n_p, damp, last):
    out_specs = [pl.BlockSpec((_R, _D), lambda i: (i, 0))]
    out_shape = [jax.ShapeDtypeStruct((n_p, _D), jnp.float32)]
    if not last:
        out_specs.append(pl.BlockSpec((4, _R, 16), lambda i: (0, i, 0)))
        out_shape.append(jax.ShapeDtypeStruct((4, n_p, 16), jnp.float32))
    return pl.pallas_call(
        functools.partial(_layer_body, float(damp), last),
        grid=(n_p // _R,),
        in_specs=[
            pl.BlockSpec((4, _R, 16), lambda i: (0, i, 0)),
            pl.BlockSpec((_R, 1), lambda i: (i, 0)),
            pl.BlockSpec((_R, 1), lambda i: (i, 0)),
            pl.BlockSpec((_R, _D), lambda i: (i, 0)),
        ],
        out_specs=out_specs,
        out_shape=out_shape,
    )


_RM = 400


def _mlp_body(au_ref, ak_ref, sa_ref, de0_ref, de1_ref, w_ref, b_ref,
              out_ref):
    deg = (de0_ref[...] + de1_ref[...]).astype(jnp.float32) + 1e-8
    ea = jnp.concatenate([sa_ref[c] for c in range(4)], axis=1) / deg
    h1 = au_ref[...]
    h2 = ak_ref[...] + ea
    w = w_ref[...]
    y = (jnp.dot(h1, w[:_D], preferred_element_type=jnp.float32)
         + jnp.dot(h2, w[_D:], preferred_element_type=jnp.float32)
         + b_ref[...])
    out_ref[...] = jnp.maximum(y, 0.0)


def _make_mlp():
    return pl.pallas_call(
        _mlp_body,
        grid=(_NE // _RM,),
        in_specs=[
            pl.BlockSpec((_RM, _D), lambda i: (i, 0)),
            pl.BlockSpec((_RM, _D), lambda i: (i, 0)),
            pl.BlockSpec((4, _RM, 16), lambda i: (0, i, 0)),
            pl.BlockSpec((_RM, 1), lambda i: (i, 0)),
            pl.BlockSpec((_RM, 1), lambda i: (i, 0)),
            pl.BlockSpec((2 * _D, _D), lambda i: (0, 0)),
            pl.BlockSpec((1, _D), lambda i: (0, 0)),
        ],
        out_specs=pl.BlockSpec((_RM, _D), lambda i: (i, 0)),
        out_shape=jax.ShapeDtypeStruct((_NE, _D), jnp.float32),
    )


def _pad_idx(x, total, dummy_base):
    pad = total - x.shape[0]
    spread = jnp.arange(pad, dtype=jnp.int32) % _SPREAD
    return jnp.concatenate([x, dummy_base + spread])


def _col4(col, total, n_tab):
    col = _pad_idx(col[0], total, col[1]) if isinstance(col, tuple) else col
    return (col[None, :]
            + (jnp.arange(4, dtype=jnp.int32) * n_tab)[:, None]).reshape(-1)


_DEG1 = _make_deg(_N1, _E1)
_DEG2 = _make_deg(_N2, _E2)
_SCAT1 = _make_scat_halves(_NUP, _EH)
_SCAT2 = _make_scat(_N2, _E2)
_SCATA = _make_scat(_NA, _EA)
_PREP1 = _make_prep(_N1)
_PREP2 = _make_prep(_N2)
_LAYER1 = (_make_layer(_N1, 2, False), _make_layer(_N1, 3, True))
_LAYER2 = (_make_layer(_N2, 2, False), _make_layer(_N2, 3, True))
_MLP = _make_mlp()


def kernel(users_feature, exercises_feature, knowledge_feature, W_mlp, b_mlp,
           ue_src, ue_dst, ek_src, ek_dst):
    zrows = jnp.zeros((1024, 16), jnp.float32)

    rowA = _pad_idx(ue_src, _EH, _NU)
    colA4 = _col4((ue_dst + _NUP, _NU), _EH, _N1)
    rowB = _pad_idx(ue_dst, _EH, _NE)
    colB4 = _col4((ue_src, _NU), _EH, _N1)
    rowcat1 = _pad_idx(jnp.concatenate([ue_src, ue_dst + _NUP]), _E1, _NU)
    feats1 = jnp.concatenate([
        jnp.pad(users_feature, ((0, _NUP - _NU), (0, 0))),
        jnp.pad(exercises_feature, ((0, _NEP - _NE), (0, 0))),
    ])

    deg1 = _DEG1(rowcat1, jnp.zeros((_N1 // 256, 16), jnp.int32),
                 jnp.arange(_N1 // 16, dtype=jnp.int32))
    d0 = deg1[0].reshape(_N1, 1)
    d1 = deg1[1].reshape(_N1, 1)
    fp = _PREP1(feats1, d0, d1)
    s = _SCAT1(fp.reshape(4 * _N1, 16), colA4, rowA, colB4, rowB,
               zrows).reshape(4, _N1, 16)
    acc1, fp = _LAYER1[0](s, d0, d1, feats1)
    s = _SCAT1(fp.reshape(4 * _N1, 16), colA4, rowA, colB4, rowB,
               zrows).reshape(4, _N1, 16)
    (acc1,) = _LAYER1[1](s, d0, d1, acc1)

    n2 = _NE + _NK
    row2 = _pad_idx(jnp.concatenate([ek_src, ek_dst + _NE]), _E2, n2)
    col2 = jnp.concatenate([ek_dst + _NE, ek_src])
    col24 = _col4((col2, n2), _E2, _N2)
    row2 = _after(row2, acc1)
    feats2 = jnp.concatenate([
        exercises_feature, knowledge_feature,
        jnp.zeros((_N2 - n2, _D), jnp.float32),
    ])

    deg2 = _DEG2(row2, jnp.zeros((_N2 // 256, 16), jnp.int32),
                 jnp.arange(_N2 // 16, dtype=jnp.int32))
    e0 = deg2[0].reshape(_N2, 1)
    e1 = deg2[1].reshape(_N2, 1)
    fp2 = _PREP2(feats2, e0, e1)
    s2 = _SCAT2(fp2.reshape(4 * _N2, 16), col24, row2, zrows)
    acc2, fp2 = _LAYER2[0](s2, e0, e1, feats2)
    s2 = _SCAT2(fp2.reshape(4 * _N2, 16), col24, row2, zrows)
    (acc2,) = _LAYER2[1](s2, e0, e1, acc2)

    krep = acc2[_NE:_NE + _NK]
    krep4 = jnp.pad(krep.reshape(_NK, 4, 16).transpose(1, 0, 2),
                    ((0, 0), (0, _NKP - _NK), (0, 0)))
    rowa = _pad_idx(ek_src, _EA, _NE)
    rowa = _after(rowa, acc2)
    cola4 = _col4((ek_dst, _NK), _EA, _NKP)
    sa = _SCATA(krep4.reshape(4 * _NKP, 16), cola4, rowa, zrows)

    a_u = lax.slice(acc1, (_NUP, 0), (_NUP + _NE, _D))
    a_k = acc2[:_NE]
    out_e = _MLP(a_u, a_k, sa, e0, e1, W_mlp, b_mlp.reshape(1, _D))
    return jnp.concatenate([acc1[:_NU], out_e], axis=0)
